# Initial kernel scaffold; baseline (speedup 1.0000x reference)
import jax
import jax.numpy as jnp
from jax.experimental import pallas as pl


def kernel(h, e, edge_index, t_emb, P_w, Q_w, R_w, en_g, en_b, ew1, eb1, ew2, eb2, tw1, tb1, tw2, tb2, U_w, V_w, nn_g, nn_b):
    raise NotImplementedError("write your pallas kernel here")


# speedup vs baseline: 3.3350x; 3.3350x over previous
import functools

import jax
import jax.numpy as jnp
from jax import lax
from jax.experimental import pallas as pl
from jax.experimental.pallas import tpu as pltpu
from jax.experimental.pallas import tpu_sc as plsc

NC = 2
NS = 16
NW = NC * NS
LANES = 16


@functools.lru_cache(maxsize=None)
def _make_gather_add(N, E, D, CH):
    per_w = E // NW
    nfull = per_w // CH
    rem = per_w - nfull * CH
    mesh = plsc.VectorSubcoreMesh(core_axis_name="c", subcore_axis_name="s")

    @functools.partial(
        pl.kernel,
        out_type=jax.ShapeDtypeStruct((E, D), jnp.float32),
        mesh=mesh,
        scratch_types=[
            pltpu.VMEM((CH,), jnp.int32),
            pltpu.VMEM((CH,), jnp.int32),
            pltpu.VMEM((CH, D), jnp.float32),
            pltpu.VMEM((CH, D), jnp.float32),
            pltpu.VMEM((max(rem, 1),), jnp.int32),
            pltpu.VMEM((max(rem, 1),), jnp.int32),
            pltpu.VMEM((max(rem, 1), D), jnp.float32),
            pltpu.VMEM((max(rem, 1), D), jnp.float32),
            pltpu.SemaphoreType.DMA,
        ],
    )
    def k(hq, hr, src, dst, g, isv, idv, qv, rv, isv2, idv2, qv2, rv2, sem):
        c = lax.axis_index("c")
        s = lax.axis_index("s")
        base = (c * NS + s) * per_w

        def do_chunk(off, n, isr, idr, qr, rr):
            pltpu.sync_copy(src.at[pl.ds(off, n)], isr)
            pltpu.sync_copy(dst.at[pl.ds(off, n)], idr)
            cq = pltpu.async_copy(hq.at[isr], qr, sem)
            cr = pltpu.async_copy(hr.at[idr], rr, sem)
            cq.wait()
            cr.wait()

            def row(i, _):
                for gi in range(D // LANES):
                    sl = pl.ds(gi * LANES, LANES)
                    qr[i, sl] = qr[i, sl] + rr[i, sl]
                return 0

            lax.fori_loop(0, n, row, 0)
            pltpu.sync_copy(qr, g.at[pl.ds(off, n)])

        def body(j, _):
            do_chunk(base + j * CH, CH, isv, idv, qv, rv)
            return 0

        lax.fori_loop(0, nfull, body, 0)
        if rem:
            do_chunk(base + nfull * CH, rem, isv2, idv2, qv2, rv2)

    return k


@functools.lru_cache(maxsize=None)
def _make_scatter_agg(N, E, D, CH):
    per_w = E // NW
    nfull = per_w // CH
    rem = per_w - nfull * CH
    rows_per_tile = ((N + NS - 1) // NS + 7) // 8 * 8
    npad = rows_per_tile * NS
    mesh = plsc.VectorSubcoreMesh(core_axis_name="c", subcore_axis_name="s")

    @functools.partial(
        pl.kernel,
        out_type=jax.ShapeDtypeStruct((NC, npad, D), jnp.float32),
        mesh=mesh,
        scratch_types=[
            pltpu.VMEM((CH,), jnp.int32),
            pltpu.VMEM((CH,), jnp.int32),
            pltpu.VMEM((CH, D), jnp.float32),
            pltpu.VMEM((CH, D), jnp.float32),
            pltpu.VMEM((max(rem, 1),), jnp.int32),
            pltpu.VMEM((max(rem, 1),), jnp.int32),
            pltpu.VMEM((max(rem, 1), D), jnp.float32),
            pltpu.VMEM((max(rem, 1), D), jnp.float32),
            pltpu.VMEM_SHARED((npad, D), jnp.float32),
            pltpu.SemaphoreType.DMA,
        ],
    )
    def k(gate, vh, src, dst, zrows, out, isv, idv, gv, vv, isv2, idv2, gv2,
          vv2, agg, sem):
        c = lax.axis_index("c")
        s = lax.axis_index("s")
        base = (c * NS + s) * per_w

        pltpu.sync_copy(zrows, agg.at[pl.ds(s * rows_per_tile, rows_per_tile)])
        plsc.subcore_barrier()

        def do_chunk(off, n, isr, idr, gr, vr):
            pltpu.sync_copy(src.at[pl.ds(off, n)], isr)
            pltpu.sync_copy(dst.at[pl.ds(off, n)], idr)
            cg = pltpu.async_copy(gate.at[pl.ds(off, n)], gr, sem)
            cv = pltpu.async_copy(vh.at[idr], vr, sem)
            cg.wait()
            cv.wait()

            def row(i, _):
                for gi in range(D // LANES):
                    sl = pl.ds(gi * LANES, LANES)
                    gr[i, sl] = gr[i, sl] * vr[i, sl]
                return 0

            lax.fori_loop(0, n, row, 0)
            pltpu.sync_copy(gr, agg.at[isr], add=True)

        def body(j, _):
            do_chunk(base + j * CH, CH, isv, idv, gv, vv)
            return 0

        lax.fori_loop(0, nfull, body, 0)
        if rem:
            do_chunk(base + nfull * CH, rem, isv2, idv2, gv2, vv2)

        plsc.subcore_barrier()
        pltpu.sync_copy(
            agg.at[pl.ds(s * rows_per_tile, rows_per_tile)],
            out.at[c, pl.ds(s * rows_per_tile, rows_per_tile)],
        )

    return k


def _node_mm(h, w4):
    N, D = h.shape
    D4 = w4.shape[1]
    nb = 10
    bl = N // nb

    def body(h_ref, w_ref, o_ref):
        o_ref[...] = jnp.dot(h_ref[...], w_ref[...],
                             preferred_element_type=jnp.float32)

    return pl.pallas_call(
        body,
        grid=(nb,),
        in_specs=[
            pl.BlockSpec((bl, D), lambda i: (i, 0)),
            pl.BlockSpec((D, D4), lambda i: (0, 0)),
        ],
        out_specs=pl.BlockSpec((bl, D4), lambda i: (i, 0)),
        out_shape=jax.ShapeDtypeStruct((N, D4), jnp.float32),
    )(h, w4)


def _ln_block(x, g, b, eps=1e-5):
    m = jnp.mean(x, axis=-1, keepdims=True)
    cx = x - m
    v = jnp.mean(cx * cx, axis=-1, keepdims=True)
    return cx / jnp.sqrt(v + eps) * g + b


def _edge_mlp(e, g, t_emb, P_w, ew1, ew2, tw1, tw2, en_g, en_b, eb1, eb2,
              tb1, tb2):
    E, D = e.shape
    eb_blk = 3200
    nb = E // eb_blk

    def body(e_ref, g_ref, t_ref, pw, w1, w2, tw1r, tw2r, eng, enb, b1, b2,
             tb1r, tb2r, enew_ref, gate_ref):
        eb = e_ref[...]
        e_hat = jnp.dot(eb, pw[...], preferred_element_type=jnp.float32) \
            + g_ref[...]
        xn = _ln_block(e_hat, eng[...], enb[...])
        h1 = jnp.maximum(
            jnp.dot(xn, w1[...], preferred_element_type=jnp.float32) + b1[...],
            0.0)
        mlp_e = jnp.dot(h1, w2[...], preferred_element_type=jnp.float32) \
            + b2[...]
        t1 = jnp.maximum(
            jnp.dot(t_ref[...], tw1r[...], preferred_element_type=jnp.float32)
            + tb1r[...], 0.0)
        mlp_t = jnp.dot(t1, tw2r[...], preferred_element_type=jnp.float32) \
            + tb2r[...]
        enew_ref[...] = eb + mlp_e + mlp_t
        gate_ref[...] = jax.nn.sigmoid(e_hat)

    full = pl.BlockSpec((D, D), lambda i: (0, 0))
    row = pl.BlockSpec((1, D), lambda i: (0, 0))
    blk = pl.BlockSpec((eb_blk, D), lambda i: (i, 0))
    return pl.pallas_call(
        body,
        grid=(nb,),
        in_specs=[blk, blk, row, full, full, full, full, full,
                  row, row, row, row, row, row],
        out_specs=[blk, blk],
        out_shape=[
            jax.ShapeDtypeStruct((E, D), jnp.float32),
            jax.ShapeDtypeStruct((E, D), jnp.float32),
        ],
    )(e, g, t_emb, P_w, ew1, ew2, tw1, tw2, en_g, en_b, eb1, eb2, tb1, tb2)


def _node_update(h, uh, a0, a1, nn_g, nn_b):
    N, D = h.shape
    nb = 10
    bl = N // nb

    def body(h_ref, uh_ref, a0_ref, a1_ref, g_ref, b_ref, o_ref):
        x = uh_ref[...] + a0_ref[...] + a1_ref[...]
        o_ref[...] = h_ref[...] + jnp.maximum(
            _ln_block(x, g_ref[...], b_ref[...]), 0.0)

    blk = pl.BlockSpec((bl, D), lambda i: (i, 0))
    row = pl.BlockSpec((1, D), lambda i: (0, 0))
    return pl.pallas_call(
        body,
        grid=(nb,),
        in_specs=[blk, blk, blk, blk, row, row],
        out_specs=blk,
        out_shape=jax.ShapeDtypeStruct((N, D), jnp.float32),
    )(h, uh, a0, a1, nn_g, nn_b)


def kernel(h, e, edge_index, t_emb, P_w, Q_w, R_w, en_g, en_b, ew1, eb1, ew2,
           eb2, tw1, tb1, tw2, tb2, U_w, V_w, nn_g, nn_b):
    N, D = h.shape
    E = e.shape[0]
    CH = 128

    src = edge_index[0]
    dst = edge_index[1]

    w4 = jnp.concatenate([Q_w, R_w, V_w, U_w], axis=1)
    nodes = _node_mm(h, w4)
    hq = nodes[:, 0:D]
    hr = nodes[:, D:2 * D]
    vh = nodes[:, 2 * D:3 * D]
    uh = nodes[:, 3 * D:4 * D]

    g = _make_gather_add(N, E, D, CH)(hq, hr, src, dst)

    e_new, gate = _edge_mlp(
        e, g, t_emb, P_w, ew1, ew2, tw1, tw2,
        en_g.reshape(1, D), en_b.reshape(1, D), eb1.reshape(1, D),
        eb2.reshape(1, D), tb1.reshape(1, D), tb2.reshape(1, D))

    rows_per_tile = ((N + NS - 1) // NS + 7) // 8 * 8
    zrows = jnp.zeros((rows_per_tile, D), jnp.float32)
    aggp = _make_scatter_agg(N, E, D, CH)(gate, vh, src, dst, zrows)

    h_new = _node_update(h, uh, aggp[0, :N], aggp[1, :N],
                         nn_g.reshape(1, D), nn_b.reshape(1, D))
    return (h_new, e_new)


# speedup vs baseline: 5.1331x; 1.5391x over previous
import functools

import jax
import jax.numpy as jnp
from jax import lax
from jax.experimental import pallas as pl
from jax.experimental.pallas import tpu as pltpu
from jax.experimental.pallas import tpu_sc as plsc

NC = 2
NS = 16
NW = NC * NS
LANES = 16


@functools.lru_cache(maxsize=None)
def _make_gather_add(N, E, D, CH):
    per_w = E // NW
    nfull = per_w // CH
    rem = per_w - nfull * CH
    assert nfull >= 2 and nfull % 2 == 0
    mesh = plsc.VectorSubcoreMesh(core_axis_name="c", subcore_axis_name="s")

    @functools.partial(
        pl.kernel,
        out_type=jax.ShapeDtypeStruct((E, D), jnp.float32),
        mesh=mesh,
        scratch_types=[
            pltpu.VMEM((per_w,), jnp.int32),
            pltpu.VMEM((per_w,), jnp.int32),
            pltpu.VMEM((CH, D), jnp.float32),
            pltpu.VMEM((CH, D), jnp.float32),
            pltpu.VMEM((CH, D), jnp.float32),
            pltpu.VMEM((CH, D), jnp.float32),
            pltpu.VMEM((max(rem, 1), D), jnp.float32),
            pltpu.VMEM((max(rem, 1), D), jnp.float32),
            pltpu.SemaphoreType.DMA,
            pltpu.SemaphoreType.DMA,
            pltpu.SemaphoreType.DMA,
            pltpu.SemaphoreType.DMA,
        ],
    )
    def k(hq, hr, src, dst, g, sall, dall, q0, r0, q1, r1, qv2, rv2,
          sg0, sg1, ss0, ss1):
        c = lax.axis_index("c")
        s = lax.axis_index("s")
        base = (c * NS + s) * per_w
        qs = (q0, q1)
        rs = (r0, r1)
        sgs = (sg0, sg1)
        sss = (ss0, ss1)

        pltpu.sync_copy(src.at[pl.ds(base, per_w)], sall)
        pltpu.sync_copy(dst.at[pl.ds(base, per_w)], dall)

        def fire(cj, b):
            isl = sall.at[pl.ds(cj * CH, CH)]
            idl = dall.at[pl.ds(cj * CH, CH)]
            pltpu.async_copy(hq.at[isl], qs[b], sgs[b])
            pltpu.async_copy(hr.at[idl], rs[b], sgs[b])

        def wait_gathers(b):
            pltpu.make_async_copy(hq.at[sall.at[pl.ds(0, CH)]], qs[b],
                                  sgs[b]).wait()
            pltpu.make_async_copy(hr.at[dall.at[pl.ds(0, CH)]], rs[b],
                                  sgs[b]).wait()

        def wait_store(b):
            pltpu.make_async_copy(qs[b], g.at[pl.ds(base, CH)], sss[b]).wait()

        def add_rows(qr, rr, n):
            def row(i, _):
                for gi in range(D // LANES):
                    sl = pl.ds(gi * LANES, LANES)
                    qr[i, sl] = qr[i, sl] + rr[i, sl]
                return 0

            lax.fori_loop(0, n, row, 0)

        def store(cj, b):
            pltpu.async_copy(qs[b], g.at[pl.ds(base + cj * CH, CH)], sss[b])

        fire(0, 0)
        fire(1, 1)
        wait_gathers(0)
        add_rows(q0, r0, CH)
        store(0, 0)

        def body(i, _):
            j = 2 * i + 1
            wait_store(0)
            fire(j + 1, 0)
            wait_gathers(1)
            add_rows(q1, r1, CH)
            store(j, 1)

            wait_store(1)
            fire(j + 2, 1)
            wait_gathers(0)
            add_rows(q0, r0, CH)
            store(j + 1, 0)
            return 0

        lax.fori_loop(0, (nfull - 2) // 2, body, 0)
        wait_store(0)
        wait_gathers(1)
        add_rows(q1, r1, CH)
        store(nfull - 1, 1)
        wait_store(1)

        if rem:
            off = nfull * CH
            isl = sall.at[pl.ds(off, rem)]
            idl = dall.at[pl.ds(off, rem)]
            cq = pltpu.async_copy(hq.at[isl], qv2, sg0)
            cr = pltpu.async_copy(hr.at[idl], rv2, sg0)
            cq.wait()
            cr.wait()
            add_rows(qv2, rv2, rem)
            pltpu.sync_copy(qv2, g.at[pl.ds(base + off, rem)])

    return k


@functools.lru_cache(maxsize=None)
def _make_scatter_agg(N, E, D, CH):
    per_w = E // NW
    nfull = per_w // CH
    rem = per_w - nfull * CH
    rows_per_tile = ((N + NS - 1) // NS + 7) // 8 * 8
    npad = rows_per_tile * NS
    mesh = plsc.VectorSubcoreMesh(core_axis_name="c", subcore_axis_name="s")

    assert nfull >= 2 and nfull % 2 == 0

    @functools.partial(
        pl.kernel,
        out_type=jax.ShapeDtypeStruct((NC, npad, D), jnp.float32),
        mesh=mesh,
        scratch_types=[
            pltpu.VMEM((per_w,), jnp.int32),
            pltpu.VMEM((CH,), jnp.int32),
            pltpu.VMEM((CH,), jnp.int32),
            pltpu.VMEM((CH, D), jnp.float32),
            pltpu.VMEM((CH, D), jnp.float32),
            pltpu.VMEM((CH, D), jnp.float32),
            pltpu.VMEM((CH, D), jnp.float32),
            pltpu.VMEM((max(rem, 1),), jnp.int32),
            pltpu.VMEM((max(rem, 1), D), jnp.float32),
            pltpu.VMEM((max(rem, 1), D), jnp.float32),
            pltpu.VMEM_SHARED((npad, D), jnp.float32),
            pltpu.SemaphoreType.DMA,
            pltpu.SemaphoreType.DMA,
            pltpu.SemaphoreType.DMA,
            pltpu.SemaphoreType.DMA,
        ],
    )
    def k(gate, vh, src, dst, zrows, out, dall, s0, s1, g0, v0, g1, v1,
          isv2, gv2, vv2, agg, sg0, sg1, ss0, ss1):
        c = lax.axis_index("c")
        s = lax.axis_index("s")
        base = (c * NS + s) * per_w
        svs = (s0, s1)
        gs = (g0, g1)
        vs = (v0, v1)
        sgs = (sg0, sg1)
        sss = (ss0, ss1)

        pltpu.sync_copy(zrows, agg.at[pl.ds(s * rows_per_tile, rows_per_tile)])
        pltpu.sync_copy(dst.at[pl.ds(base, per_w)], dall)
        plsc.subcore_barrier()

        def fire(cj, b):
            off = base + cj * CH
            idl = dall.at[pl.ds(cj * CH, CH)]
            pltpu.async_copy(src.at[pl.ds(off, CH)], svs[b], sgs[b])
            pltpu.async_copy(gate.at[pl.ds(off, CH)], gs[b], sgs[b])
            pltpu.async_copy(vh.at[idl], vs[b], sgs[b])

        def wait_fire(b):
            pltpu.make_async_copy(src.at[pl.ds(base, CH)], svs[b],
                                  sgs[b]).wait()
            pltpu.make_async_copy(gate.at[pl.ds(base, CH)], gs[b],
                                  sgs[b]).wait()
            pltpu.make_async_copy(vh.at[dall.at[pl.ds(0, CH)]], vs[b],
                                  sgs[b]).wait()

        def mul_rows(gr, vr, n):
            def row(i, _):
                for gi in range(D // LANES):
                    sl = pl.ds(gi * LANES, LANES)
                    gr[i, sl] = gr[i, sl] * vr[i, sl]
                return 0

            lax.fori_loop(0, n, row, 0)

        def scatter(b):
            pltpu.async_copy(gs[b], agg.at[svs[b]], sss[b], add=True)

        def wait_scatter(b):
            pltpu.make_async_copy(gs[b], agg.at[svs[b]], sss[b]).wait()

        fire(0, 0)
        fire(1, 1)
        wait_fire(0)
        mul_rows(g0, v0, CH)
        scatter(0)

        def body(i, _):
            j = 2 * i + 1
            wait_scatter(0)
            fire(j + 1, 0)
            wait_fire(1)
            mul_rows(g1, v1, CH)
            scatter(1)

            wait_scatter(1)
            fire(j + 2, 1)
            wait_fire(0)
            mul_rows(g0, v0, CH)
            scatter(0)
            return 0

        lax.fori_loop(0, (nfull - 2) // 2, body, 0)
        wait_scatter(0)
        wait_fire(1)
        mul_rows(g1, v1, CH)
        scatter(1)
        wait_scatter(1)

        if rem:
            off = base + nfull * CH
            ci = pltpu.async_copy(src.at[pl.ds(off, rem)], isv2, sg0)
            cg = pltpu.async_copy(gate.at[pl.ds(off, rem)], gv2, sg0)
            cv = pltpu.async_copy(vh.at[dall.at[pl.ds(nfull * CH, rem)]],
                                  vv2, sg0)
            ci.wait()
            cg.wait()
            cv.wait()
            mul_rows(gv2, vv2, rem)
            pltpu.sync_copy(gv2, agg.at[isv2], add=True)

        plsc.subcore_barrier()
        pltpu.sync_copy(
            agg.at[pl.ds(s * rows_per_tile, rows_per_tile)],
            out.at[c, pl.ds(s * rows_per_tile, rows_per_tile)],
        )

    return k


def _node_mm(h, w4):
    N, D = h.shape
    D4 = w4.shape[1]
    nb = 10
    bl = N // nb

    def body(h_ref, w_ref, o_ref):
        o_ref[...] = jnp.dot(h_ref[...], w_ref[...],
                             preferred_element_type=jnp.float32)

    return pl.pallas_call(
        body,
        grid=(nb,),
        in_specs=[
            pl.BlockSpec((bl, D), lambda i: (i, 0)),
            pl.BlockSpec((D, D4), lambda i: (0, 0)),
        ],
        out_specs=pl.BlockSpec((bl, D4), lambda i: (i, 0)),
        out_shape=jax.ShapeDtypeStruct((N, D4), jnp.float32),
    )(h, w4)


def _ln_block(x, g, b, eps=1e-5):
    m = jnp.mean(x, axis=-1, keepdims=True)
    cx = x - m
    v = jnp.mean(cx * cx, axis=-1, keepdims=True)
    return cx / jnp.sqrt(v + eps) * g + b


def _edge_mlp(e, g, t_emb, P_w, ew1, ew2, tw1, tw2, en_g, en_b, eb1, eb2,
              tb1, tb2):
    E, D = e.shape
    eb_blk = 3200
    nb = E // eb_blk

    def body(e_ref, g_ref, t_ref, pw, w1, w2, tw1r, tw2r, eng, enb, b1, b2,
             tb1r, tb2r, enew_ref, gate_ref):
        eb = e_ref[...]
        e_hat = jnp.dot(eb, pw[...], preferred_element_type=jnp.float32) \
            + g_ref[...]
        xn = _ln_block(e_hat, eng[...], enb[...])
        h1 = jnp.maximum(
            jnp.dot(xn, w1[...], preferred_element_type=jnp.float32) + b1[...],
            0.0)
        mlp_e = jnp.dot(h1, w2[...], preferred_element_type=jnp.float32) \
            + b2[...]
        t1 = jnp.maximum(
            jnp.dot(t_ref[...], tw1r[...], preferred_element_type=jnp.float32)
            + tb1r[...], 0.0)
        mlp_t = jnp.dot(t1, tw2r[...], preferred_element_type=jnp.float32) \
            + tb2r[...]
        enew_ref[...] = eb + mlp_e + mlp_t
        gate_ref[...] = jax.nn.sigmoid(e_hat)

    full = pl.BlockSpec((D, D), lambda i: (0, 0))
    row = pl.BlockSpec((1, D), lambda i: (0, 0))
    blk = pl.BlockSpec((eb_blk, D), lambda i: (i, 0))
    return pl.pallas_call(
        body,
        grid=(nb,),
        in_specs=[blk, blk, row, full, full, full, full, full,
                  row, row, row, row, row, row],
        out_specs=[blk, blk],
        out_shape=[
            jax.ShapeDtypeStruct((E, D), jnp.float32),
            jax.ShapeDtypeStruct((E, D), jnp.float32),
        ],
    )(e, g, t_emb, P_w, ew1, ew2, tw1, tw2, en_g, en_b, eb1, eb2, tb1, tb2)


def _node_update(h, uh, a0, a1, nn_g, nn_b):
    N, D = h.shape
    nb = 10
    bl = N // nb

    def body(h_ref, uh_ref, a0_ref, a1_ref, g_ref, b_ref, o_ref):
        x = uh_ref[...] + a0_ref[...] + a1_ref[...]
        o_ref[...] = h_ref[...] + jnp.maximum(
            _ln_block(x, g_ref[...], b_ref[...]), 0.0)

    blk = pl.BlockSpec((bl, D), lambda i: (i, 0))
    row = pl.BlockSpec((1, D), lambda i: (0, 0))
    return pl.pallas_call(
        body,
        grid=(nb,),
        in_specs=[blk, blk, blk, blk, row, row],
        out_specs=blk,
        out_shape=jax.ShapeDtypeStruct((N, D), jnp.float32),
    )(h, uh, a0, a1, nn_g, nn_b)


def kernel(h, e, edge_index, t_emb, P_w, Q_w, R_w, en_g, en_b, ew1, eb1, ew2,
           eb2, tw1, tb1, tw2, tb2, U_w, V_w, nn_g, nn_b):
    N, D = h.shape
    E = e.shape[0]
    CH = 128

    src = edge_index[0]
    dst = edge_index[1]

    w4 = jnp.concatenate([Q_w, R_w, V_w, U_w], axis=1)
    nodes = _node_mm(h, w4)
    hq = nodes[:, 0:D]
    hr = nodes[:, D:2 * D]
    vh = nodes[:, 2 * D:3 * D]
    uh = nodes[:, 3 * D:4 * D]

    g = _make_gather_add(N, E, D, 128)(hq, hr, src, dst)

    e_new, gate = _edge_mlp(
        e, g, t_emb, P_w, ew1, ew2, tw1, tw2,
        en_g.reshape(1, D), en_b.reshape(1, D), eb1.reshape(1, D),
        eb2.reshape(1, D), tb1.reshape(1, D), tb2.reshape(1, D))

    rows_per_tile = ((N + NS - 1) // NS + 7) // 8 * 8
    zrows = jnp.zeros((rows_per_tile, D), jnp.float32)
    aggp = _make_scatter_agg(N, E, D, 64)(gate, vh, src, dst, zrows)

    h_new = _node_update(h, uh, aggp[0, :N], aggp[1, :N],
                         nn_g.reshape(1, D), nn_b.reshape(1, D))
    return (h_new, e_new)


# speedup vs baseline: 5.6393x; 1.0986x over previous
import functools

import jax
import jax.numpy as jnp
from jax import lax
from jax.experimental import pallas as pl
from jax.experimental.pallas import tpu as pltpu
from jax.experimental.pallas import tpu_sc as plsc

NC = 2
NS = 16
NW = NC * NS
LANES = 16


@functools.lru_cache(maxsize=None)
def _make_gather_add(N, E, D, CH):
    per_w = E // NW
    nfull = per_w // CH
    rem = per_w - nfull * CH
    assert nfull >= 2 and nfull % 2 == 0
    mesh = plsc.VectorSubcoreMesh(core_axis_name="c", subcore_axis_name="s")

    @functools.partial(
        pl.kernel,
        out_type=jax.ShapeDtypeStruct((E, D), jnp.float32),
        mesh=mesh,
        scratch_types=[
            pltpu.VMEM((per_w,), jnp.int32),
            pltpu.VMEM((per_w,), jnp.int32),
            pltpu.VMEM((CH, D), jnp.float32),
            pltpu.VMEM((CH, D), jnp.float32),
            pltpu.VMEM((CH, D), jnp.float32),
            pltpu.VMEM((CH, D), jnp.float32),
            pltpu.VMEM((max(rem, 1), D), jnp.float32),
            pltpu.VMEM((max(rem, 1), D), jnp.float32),
            pltpu.SemaphoreType.DMA,
            pltpu.SemaphoreType.DMA,
            pltpu.SemaphoreType.DMA,
            pltpu.SemaphoreType.DMA,
        ],
    )
    def k(hq, hr, src, dst, g, sall, dall, q0, r0, q1, r1, qv2, rv2,
          sg0, sg1, ss0, ss1):
        c = lax.axis_index("c")
        s = lax.axis_index("s")
        base = (c * NS + s) * per_w
        qs = (q0, q1)
        rs = (r0, r1)
        sgs = (sg0, sg1)
        sss = (ss0, ss1)

        pltpu.sync_copy(src.at[pl.ds(base, per_w)], sall)
        pltpu.sync_copy(dst.at[pl.ds(base, per_w)], dall)

        def fire(cj, b):
            isl = sall.at[pl.ds(cj * CH, CH)]
            idl = dall.at[pl.ds(cj * CH, CH)]
            pltpu.async_copy(hq.at[isl], qs[b], sgs[b])
            pltpu.async_copy(hr.at[idl], rs[b], sgs[b])

        def wait_gathers(b):
            pltpu.make_async_copy(hq.at[sall.at[pl.ds(0, CH)]], qs[b],
                                  sgs[b]).wait()
            pltpu.make_async_copy(hr.at[dall.at[pl.ds(0, CH)]], rs[b],
                                  sgs[b]).wait()

        def wait_store(b):
            pltpu.make_async_copy(qs[b], g.at[pl.ds(base, CH)], sss[b]).wait()

        def add_rows(qr, rr, n):
            def row(i, _):
                for gi in range(D // LANES):
                    sl = pl.ds(gi * LANES, LANES)
                    qr[i, sl] = qr[i, sl] + rr[i, sl]
                return 0

            lax.fori_loop(0, n, row, 0)

        def store(cj, b):
            pltpu.async_copy(qs[b], g.at[pl.ds(base + cj * CH, CH)], sss[b])

        fire(0, 0)
        fire(1, 1)
        wait_gathers(0)
        add_rows(q0, r0, CH)
        store(0, 0)

        def body(i, _):
            j = 2 * i + 1
            wait_store(0)
            fire(j + 1, 0)
            wait_gathers(1)
            add_rows(q1, r1, CH)
            store(j, 1)

            wait_store(1)
            fire(j + 2, 1)
            wait_gathers(0)
            add_rows(q0, r0, CH)
            store(j + 1, 0)
            return 0

        lax.fori_loop(0, (nfull - 2) // 2, body, 0)
        wait_store(0)
        wait_gathers(1)
        add_rows(q1, r1, CH)
        store(nfull - 1, 1)
        wait_store(1)

        if rem:
            off = nfull * CH
            isl = sall.at[pl.ds(off, rem)]
            idl = dall.at[pl.ds(off, rem)]
            cq = pltpu.async_copy(hq.at[isl], qv2, sg0)
            cr = pltpu.async_copy(hr.at[idl], rv2, sg0)
            cq.wait()
            cr.wait()
            add_rows(qv2, rv2, rem)
            pltpu.sync_copy(qv2, g.at[pl.ds(base + off, rem)])

    return k


@functools.lru_cache(maxsize=None)
def _make_scatter_agg(N, E, D, CH):
    per_w = E // NW
    nfull = per_w // CH
    rem = per_w - nfull * CH
    rows_per_tile = ((N + NS - 1) // NS + 7) // 8 * 8
    npad = rows_per_tile * NS
    mesh = plsc.VectorSubcoreMesh(core_axis_name="c", subcore_axis_name="s")

    assert nfull >= 2 and nfull % 2 == 0

    @functools.partial(
        pl.kernel,
        out_type=jax.ShapeDtypeStruct((NC, npad, D), jnp.float32),
        mesh=mesh,
        scratch_types=[
            pltpu.VMEM((per_w,), jnp.int32),
            pltpu.VMEM((CH,), jnp.int32),
            pltpu.VMEM((CH,), jnp.int32),
            pltpu.VMEM((CH, D), jnp.float32),
            pltpu.VMEM((CH, D), jnp.float32),
            pltpu.VMEM((CH, D), jnp.float32),
            pltpu.VMEM((CH, D), jnp.float32),
            pltpu.VMEM((max(rem, 1),), jnp.int32),
            pltpu.VMEM((max(rem, 1), D), jnp.float32),
            pltpu.VMEM((max(rem, 1), D), jnp.float32),
            pltpu.VMEM_SHARED((npad, D), jnp.float32),
            pltpu.SemaphoreType.DMA,
            pltpu.SemaphoreType.DMA,
            pltpu.SemaphoreType.DMA,
            pltpu.SemaphoreType.DMA,
        ],
    )
    def k(gate, vh, src, dst, zrows, out, dall, s0, s1, g0, v0, g1, v1,
          isv2, gv2, vv2, agg, sg0, sg1, ss0, ss1):
        c = lax.axis_index("c")
        s = lax.axis_index("s")
        base = (c * NS + s) * per_w
        svs = (s0, s1)
        gs = (g0, g1)
        vs = (v0, v1)
        sgs = (sg0, sg1)
        sss = (ss0, ss1)

        pltpu.sync_copy(zrows, agg.at[pl.ds(s * rows_per_tile, rows_per_tile)])
        pltpu.sync_copy(dst.at[pl.ds(base, per_w)], dall)
        plsc.subcore_barrier()

        def fire(cj, b):
            off = base + cj * CH
            idl = dall.at[pl.ds(cj * CH, CH)]
            pltpu.async_copy(src.at[pl.ds(off, CH)], svs[b], sgs[b])
            pltpu.async_copy(gate.at[pl.ds(off, CH)], gs[b], sgs[b])
            pltpu.async_copy(vh.at[idl], vs[b], sgs[b])

        def wait_fire(b):
            pltpu.make_async_copy(src.at[pl.ds(base, CH)], svs[b],
                                  sgs[b]).wait()
            pltpu.make_async_copy(gate.at[pl.ds(base, CH)], gs[b],
                                  sgs[b]).wait()
            pltpu.make_async_copy(vh.at[dall.at[pl.ds(0, CH)]], vs[b],
                                  sgs[b]).wait()

        def mul_rows(gr, vr, n):
            def row(i, _):
                for gi in range(D // LANES):
                    sl = pl.ds(gi * LANES, LANES)
                    gr[i, sl] = gr[i, sl] * vr[i, sl]
                return 0

            lax.fori_loop(0, n, row, 0)

        def scatter(b):
            pltpu.async_copy(gs[b], agg.at[svs[b]], sss[b], add=True)

        def wait_scatter(b):
            pltpu.make_async_copy(gs[b], agg.at[svs[b]], sss[b]).wait()

        fire(0, 0)
        fire(1, 1)
        wait_fire(0)
        mul_rows(g0, v0, CH)
        scatter(0)

        def body(i, _):
            j = 2 * i + 1
            wait_scatter(0)
            fire(j + 1, 0)
            wait_fire(1)
            mul_rows(g1, v1, CH)
            scatter(1)

            wait_scatter(1)
            fire(j + 2, 1)
            wait_fire(0)
            mul_rows(g0, v0, CH)
            scatter(0)
            return 0

        lax.fori_loop(0, (nfull - 2) // 2, body, 0)
        wait_scatter(0)
        wait_fire(1)
        mul_rows(g1, v1, CH)
        scatter(1)
        wait_scatter(1)

        if rem:
            off = base + nfull * CH
            ci = pltpu.async_copy(src.at[pl.ds(off, rem)], isv2, sg0)
            cg = pltpu.async_copy(gate.at[pl.ds(off, rem)], gv2, sg0)
            cv = pltpu.async_copy(vh.at[dall.at[pl.ds(nfull * CH, rem)]],
                                  vv2, sg0)
            ci.wait()
            cg.wait()
            cv.wait()
            mul_rows(gv2, vv2, rem)
            pltpu.sync_copy(gv2, agg.at[isv2], add=True)

        plsc.subcore_barrier()
        pltpu.sync_copy(
            agg.at[pl.ds(s * rows_per_tile, rows_per_tile)],
            out.at[c, pl.ds(s * rows_per_tile, rows_per_tile)],
        )

    return k


def _node_mm(h, w4):
    N, D = h.shape
    D4 = w4.shape[1]
    nb = 10
    bl = N // nb

    def body(h_ref, w_ref, o_ref):
        o_ref[...] = jnp.dot(h_ref[...], w_ref[...],
                             preferred_element_type=jnp.float32)

    return pl.pallas_call(
        body,
        grid=(nb,),
        in_specs=[
            pl.BlockSpec((bl, D), lambda i: (i, 0)),
            pl.BlockSpec((D, D4), lambda i: (0, 0)),
        ],
        out_specs=pl.BlockSpec((bl, D4), lambda i: (i, 0)),
        out_shape=jax.ShapeDtypeStruct((N, D4), jnp.float32),
    )(h, w4)


def _ln_block(x, g, b, eps=1e-5):
    m = jnp.mean(x, axis=-1, keepdims=True)
    cx = x - m
    v = jnp.mean(cx * cx, axis=-1, keepdims=True)
    return cx / jnp.sqrt(v + eps) * g + b


def _edge_mlp(e, g, t_emb, P_w, ew1, ew2, tw1, tw2, en_g, en_b, eb1, eb2,
              tb1, tb2, blk_off, e_new_prev):
    E, D = e.shape
    Ep = g.shape[0]
    eb_blk = 2560
    nb = Ep // eb_blk

    def body(e_ref, g_ref, t_ref, pw, w1, w2, tw1r, tw2r, eng, enb, b1, b2,
             tb1r, tb2r, *rest):
        enew_ref, gate_ref = rest[-2], rest[-1]
        eb = e_ref[...]
        e_hat = jnp.dot(eb, pw[...], preferred_element_type=jnp.float32) \
            + g_ref[...]
        xn = _ln_block(e_hat, eng[...], enb[...])
        h1 = jnp.maximum(
            jnp.dot(xn, w1[...], preferred_element_type=jnp.float32) + b1[...],
            0.0)
        mlp_e = jnp.dot(h1, w2[...], preferred_element_type=jnp.float32) \
            + b2[...]
        t1 = jnp.maximum(
            jnp.dot(t_ref[...], tw1r[...], preferred_element_type=jnp.float32)
            + tb1r[...], 0.0)
        mlp_t = jnp.dot(t1, tw2r[...], preferred_element_type=jnp.float32) \
            + tb2r[...]
        enew_ref[...] = eb + mlp_e + mlp_t
        gate_ref[...] = jax.nn.sigmoid(e_hat)

    full = pl.BlockSpec((D, D), lambda i: (0, 0))
    row = pl.BlockSpec((1, D), lambda i: (0, 0))
    blk = pl.BlockSpec((eb_blk, D), lambda i: (i, 0))
    off_blk = pl.BlockSpec((eb_blk, D), lambda i: (i + blk_off, 0))
    in_specs = [off_blk, blk, row, full, full, full, full, full,
                row, row, row, row, row, row]
    operands = [e, g, t_emb, P_w, ew1, ew2, tw1, tw2, en_g, en_b, eb1, eb2,
                tb1, tb2]
    aliases = {}
    if e_new_prev is not None:
        in_specs.append(pl.BlockSpec((8, D), lambda i: (0, 0)))
        operands.append(e_new_prev)
        aliases = {14: 0}
    return pl.pallas_call(
        body,
        grid=(nb,),
        in_specs=in_specs,
        out_specs=[off_blk, blk],
        out_shape=[
            jax.ShapeDtypeStruct((E, D), jnp.float32),
            jax.ShapeDtypeStruct((Ep, D), jnp.float32),
        ],
        input_output_aliases=aliases,
    )(*operands)


def _node_update(h, uh, parts, nn_g, nn_b):
    N, D = h.shape
    nb = 10
    bl = N // nb
    np_ = len(parts)

    def body(h_ref, uh_ref, *rest):
        a_refs = rest[:np_]
        g_ref, b_ref, o_ref = rest[np_], rest[np_ + 1], rest[np_ + 2]
        x = uh_ref[...]
        for a in a_refs:
            x = x + a[...]
        o_ref[...] = h_ref[...] + jnp.maximum(
            _ln_block(x, g_ref[...], b_ref[...]), 0.0)

    blk = pl.BlockSpec((bl, D), lambda i: (i, 0))
    row = pl.BlockSpec((1, D), lambda i: (0, 0))
    return pl.pallas_call(
        body,
        grid=(nb,),
        in_specs=[blk, blk] + [blk] * np_ + [row, row],
        out_specs=blk,
        out_shape=jax.ShapeDtypeStruct((N, D), jnp.float32),
    )(h, uh, *parts, nn_g, nn_b)


def kernel(h, e, edge_index, t_emb, P_w, Q_w, R_w, en_g, en_b, ew1, eb1, ew2,
           eb2, tw1, tb1, tw2, tb2, U_w, V_w, nn_g, nn_b):
    N, D = h.shape
    E = e.shape[0]
    EB = 2560

    src = edge_index[0]
    dst = edge_index[1]

    w4 = jnp.concatenate([Q_w, R_w, V_w, U_w], axis=1)
    nodes = _node_mm(h, w4)
    hq = nodes[:, 0:D]
    hr = nodes[:, D:2 * D]
    vh = nodes[:, 2 * D:3 * D]
    uh = nodes[:, 3 * D:4 * D]

    E0 = (E // 2 + NW * 128 - 1) // (NW * 128) * (NW * 128)
    assert E0 % EB == 0 and (E - E0) % EB == 0
    bounds = [(0, E0), (E0, E)]

    rows_per_tile = ((N + NS - 1) // NS + 7) // 8 * 8
    zrows = jnp.zeros((rows_per_tile, D), jnp.float32)

    gs = []
    for lo, hi in bounds:
        gs.append(_make_gather_add(N, hi - lo, D, 128)(
            hq, hr, src[lo:hi], dst[lo:hi]))

    e_new = None
    gates = []
    for (lo, hi), g in zip(bounds, gs):
        e_new, gate = _edge_mlp(
            e, g, t_emb, P_w, ew1, ew2, tw1, tw2,
            en_g.reshape(1, D), en_b.reshape(1, D), eb1.reshape(1, D),
            eb2.reshape(1, D), tb1.reshape(1, D), tb2.reshape(1, D),
            lo // EB, e_new)
        gates.append(gate)

    parts = []
    for (lo, hi), gate in zip(bounds, gates):
        aggp = _make_scatter_agg(N, hi - lo, D, 64)(
            gate, vh, src[lo:hi], dst[lo:hi], zrows)
        parts.extend([aggp[0, :N], aggp[1, :N]])

    h_new = _node_update(h, uh, parts, nn_g.reshape(1, D), nn_b.reshape(1, D))
    return (h_new, e_new)


# speedup vs baseline: 5.7669x; 1.0226x over previous
import functools

import jax
import jax.numpy as jnp
from jax import lax
from jax.experimental import pallas as pl
from jax.experimental.pallas import tpu as pltpu
from jax.experimental.pallas import tpu_sc as plsc

NC = 2
NS = 16
NW = NC * NS
LANES = 16


@functools.lru_cache(maxsize=None)
def _make_gather_add(N, E, D, CH):
    per_w = E // NW
    nfull = per_w // CH
    rem = per_w - nfull * CH
    assert nfull >= 2 and nfull % 2 == 0
    mesh = plsc.VectorSubcoreMesh(core_axis_name="c", subcore_axis_name="s")

    @functools.partial(
        pl.kernel,
        out_type=jax.ShapeDtypeStruct((E, D), jnp.float32),
        mesh=mesh,
        scratch_types=[
            pltpu.VMEM((per_w,), jnp.int32),
            pltpu.VMEM((per_w,), jnp.int32),
            pltpu.VMEM((CH, D), jnp.float32),
            pltpu.VMEM((CH, D), jnp.float32),
            pltpu.VMEM((CH, D), jnp.float32),
            pltpu.VMEM((CH, D), jnp.float32),
            pltpu.VMEM((max(rem, 1), D), jnp.float32),
            pltpu.VMEM((max(rem, 1), D), jnp.float32),
            pltpu.SemaphoreType.DMA,
            pltpu.SemaphoreType.DMA,
            pltpu.SemaphoreType.DMA,
            pltpu.SemaphoreType.DMA,
        ],
    )
    def k(hq, hr, src, dst, g, sall, dall, q0, r0, q1, r1,
          qv2, rv2, sg0, sg1, ss0, ss1):
        c = lax.axis_index("c")
        s = lax.axis_index("s")
        base = (c * NS + s) * per_w
        qs = (q0, q1)
        rs = (r0, r1)
        sgs = (sg0, sg1)
        sss = (ss0, ss1)

        pltpu.sync_copy(src.at[pl.ds(base, per_w)], sall)
        pltpu.sync_copy(dst.at[pl.ds(base, per_w)], dall)

        def fire(cj, b):
            isl = sall.at[pl.ds(cj * CH, CH)]
            idl = dall.at[pl.ds(cj * CH, CH)]
            pltpu.async_copy(hq.at[isl], qs[b], sgs[b])
            pltpu.async_copy(hr.at[idl], rs[b], sgs[b])

        def wait_gathers(b):
            pltpu.make_async_copy(hq.at[sall.at[pl.ds(0, CH)]], qs[b],
                                  sgs[b]).wait()
            pltpu.make_async_copy(hr.at[dall.at[pl.ds(0, CH)]], rs[b],
                                  sgs[b]).wait()

        def wait_store(b):
            pltpu.make_async_copy(qs[b], g.at[pl.ds(base, CH)], sss[b]).wait()

        def add_rows(qr, rr, n):
            def row(i, _):
                for gi in range(D // LANES):
                    sl = pl.ds(gi * LANES, LANES)
                    qr[i, sl] = qr[i, sl] + rr[i, sl]
                return 0

            lax.fori_loop(0, n, row, 0)

        def store(cj, b):
            pltpu.async_copy(qs[b], g.at[pl.ds(base + cj * CH, CH)], sss[b])

        fire(0, 0)
        fire(1, 1)
        wait_gathers(0)
        add_rows(q0, r0, CH)
        store(0, 0)

        def body(i, _):
            j = 2 * i + 1
            wait_store(0)
            fire(j + 1, 0)
            wait_gathers(1)
            add_rows(q1, r1, CH)
            store(j, 1)

            wait_store(1)
            fire(j + 2, 1)
            wait_gathers(0)
            add_rows(q0, r0, CH)
            store(j + 1, 0)
            return 0

        lax.fori_loop(0, (nfull - 2) // 2, body, 0)
        wait_store(0)
        wait_gathers(1)
        add_rows(q1, r1, CH)
        store(nfull - 1, 1)
        wait_store(1)

        if rem:
            off = nfull * CH
            isl = sall.at[pl.ds(off, rem)]
            idl = dall.at[pl.ds(off, rem)]
            cq = pltpu.async_copy(hq.at[isl], qv2, sg0)
            cr = pltpu.async_copy(hr.at[idl], rv2, sg0)
            cq.wait()
            cr.wait()
            add_rows(qv2, rv2, rem)
            pltpu.sync_copy(qv2, g.at[pl.ds(base + off, rem)])

    return k


@functools.lru_cache(maxsize=None)
def _make_scatter_agg(N, E, D, CH):
    per_w = E // NW
    nfull = per_w // CH
    rem = per_w - nfull * CH
    rows_per_tile = ((N + NS - 1) // NS + 7) // 8 * 8
    npad = rows_per_tile * NS
    mesh = plsc.VectorSubcoreMesh(core_axis_name="c", subcore_axis_name="s")

    assert nfull >= 2 and nfull % 2 == 0

    @functools.partial(
        pl.kernel,
        out_type=jax.ShapeDtypeStruct((NC, npad, D), jnp.float32),
        mesh=mesh,
        scratch_types=[
            pltpu.VMEM((per_w,), jnp.int32),
            pltpu.VMEM((CH,), jnp.int32),
            pltpu.VMEM((CH,), jnp.int32),
            pltpu.VMEM((CH, D), jnp.float32),
            pltpu.VMEM((CH, D), jnp.float32),
            pltpu.VMEM((CH, D), jnp.float32),
            pltpu.VMEM((CH, D), jnp.float32),
            pltpu.VMEM((max(rem, 1),), jnp.int32),
            pltpu.VMEM((max(rem, 1), D), jnp.float32),
            pltpu.VMEM((max(rem, 1), D), jnp.float32),
            pltpu.VMEM_SHARED((npad, D), jnp.float32),
            pltpu.SemaphoreType.DMA,
            pltpu.SemaphoreType.DMA,
            pltpu.SemaphoreType.DMA,
            pltpu.SemaphoreType.DMA,
        ],
    )
    def k(gate, vh, src, dst, zrows, out, dall, s0, s1, g0, v0, g1, v1,
          isv2, gv2, vv2, agg, sg0, sg1, ss0, ss1):
        c = lax.axis_index("c")
        s = lax.axis_index("s")
        base = (c * NS + s) * per_w
        svs = (s0, s1)
        gs = (g0, g1)
        vs = (v0, v1)
        sgs = (sg0, sg1)
        sss = (ss0, ss1)

        pltpu.sync_copy(zrows, agg.at[pl.ds(s * rows_per_tile, rows_per_tile)])
        pltpu.sync_copy(dst.at[pl.ds(base, per_w)], dall)
        plsc.subcore_barrier()

        def fire(cj, b):
            off = base + cj * CH
            idl = dall.at[pl.ds(cj * CH, CH)]
            pltpu.async_copy(src.at[pl.ds(off, CH)], svs[b], sgs[b])
            pltpu.async_copy(gate.at[pl.ds(off, CH)], gs[b], sgs[b])
            pltpu.async_copy(vh.at[idl], vs[b], sgs[b])

        def wait_fire(b):
            pltpu.make_async_copy(src.at[pl.ds(base, CH)], svs[b],
                                  sgs[b]).wait()
            pltpu.make_async_copy(gate.at[pl.ds(base, CH)], gs[b],
                                  sgs[b]).wait()
            pltpu.make_async_copy(vh.at[dall.at[pl.ds(0, CH)]], vs[b],
                                  sgs[b]).wait()

        def mul_rows(gr, vr, n):
            def row(i, _):
                for gi in range(D // LANES):
                    sl = pl.ds(gi * LANES, LANES)
                    gr[i, sl] = gr[i, sl] * vr[i, sl]
                return 0

            lax.fori_loop(0, n, row, 0)

        def scatter(b):
            pltpu.async_copy(gs[b], agg.at[svs[b]], sss[b], add=True)

        def wait_scatter(b):
            pltpu.make_async_copy(gs[b], agg.at[svs[b]], sss[b]).wait()

        fire(0, 0)
        fire(1, 1)
        wait_fire(0)
        mul_rows(g0, v0, CH)
        scatter(0)

        def body(i, _):
            j = 2 * i + 1
            wait_scatter(0)
            fire(j + 1, 0)
            wait_fire(1)
            mul_rows(g1, v1, CH)
            scatter(1)

            wait_scatter(1)
            fire(j + 2, 1)
            wait_fire(0)
            mul_rows(g0, v0, CH)
            scatter(0)
            return 0

        lax.fori_loop(0, (nfull - 2) // 2, body, 0)
        wait_scatter(0)
        wait_fire(1)
        mul_rows(g1, v1, CH)
        scatter(1)
        wait_scatter(1)

        if rem:
            off = base + nfull * CH
            ci = pltpu.async_copy(src.at[pl.ds(off, rem)], isv2, sg0)
            cg = pltpu.async_copy(gate.at[pl.ds(off, rem)], gv2, sg0)
            cv = pltpu.async_copy(vh.at[dall.at[pl.ds(nfull * CH, rem)]],
                                  vv2, sg0)
            ci.wait()
            cg.wait()
            cv.wait()
            mul_rows(gv2, vv2, rem)
            pltpu.sync_copy(gv2, agg.at[isv2], add=True)

        plsc.subcore_barrier()
        pltpu.sync_copy(
            agg.at[pl.ds(s * rows_per_tile, rows_per_tile)],
            out.at[c, pl.ds(s * rows_per_tile, rows_per_tile)],
        )

    return k


def _node_mm(h, w4):
    N, D = h.shape
    D4 = w4.shape[1]
    nb = 10
    bl = N // nb

    def body(h_ref, w_ref, hq_ref, hr_ref, vh_ref, uh_ref):
        z = jnp.dot(h_ref[...], w_ref[...], preferred_element_type=jnp.float32)
        hq_ref[...] = z[:, 0:D]
        hr_ref[...] = z[:, D:2 * D]
        vh_ref[...] = z[:, 2 * D:3 * D]
        uh_ref[...] = z[:, 3 * D:4 * D]

    blk = pl.BlockSpec((bl, D), lambda i: (i, 0))
    sds = jax.ShapeDtypeStruct((N, D), jnp.float32)
    return pl.pallas_call(
        body,
        grid=(nb,),
        in_specs=[
            pl.BlockSpec((bl, D), lambda i: (i, 0)),
            pl.BlockSpec((D, D4), lambda i: (0, 0)),
        ],
        out_specs=[blk, blk, blk, blk],
        out_shape=[sds, sds, sds, sds],
    )(h, w4)


def _ln_block(x, g, b, eps=1e-5):
    m = jnp.mean(x, axis=-1, keepdims=True)
    cx = x - m
    v = jnp.mean(cx * cx, axis=-1, keepdims=True)
    return cx / jnp.sqrt(v + eps) * g + b


def _edge_mlp(e, g, t_emb, P_w, ew1, ew2, tw1, tw2, en_g, en_b, eb1, eb2,
              tb1, tb2, blk_off, e_new_prev):
    E, D = e.shape
    Ep = g.shape[0]
    eb_blk = 2560
    nb = Ep // eb_blk

    def body(e_ref, g_ref, t_ref, pw, w1, w2, tw1r, tw2r, eng, enb,
             b1, b2, tb1r, tb2r, *rest):
        enew_ref, gate_ref = rest[-2], rest[-1]
        eb = e_ref[...]
        e_hat = jnp.dot(eb, pw[...], preferred_element_type=jnp.float32) \
            + g_ref[...]
        xn = _ln_block(e_hat, eng[...], enb[...])
        h1 = jnp.maximum(
            jnp.dot(xn, w1[...], preferred_element_type=jnp.float32) + b1[...],
            0.0)
        mlp_e = jnp.dot(h1, w2[...], preferred_element_type=jnp.float32) \
            + b2[...]
        t1 = jnp.maximum(
            jnp.dot(t_ref[...], tw1r[...], preferred_element_type=jnp.float32)
            + tb1r[...], 0.0)
        mlp_t = jnp.dot(t1, tw2r[...], preferred_element_type=jnp.float32) \
            + tb2r[...]
        enew_ref[...] = eb + mlp_e + mlp_t
        gate_ref[...] = jax.nn.sigmoid(e_hat)

    full = pl.BlockSpec((D, D), lambda i: (0, 0))
    row = pl.BlockSpec((1, D), lambda i: (0, 0))
    blk = pl.BlockSpec((eb_blk, D), lambda i: (i, 0))
    off_blk = pl.BlockSpec((eb_blk, D), lambda i: (i + blk_off, 0))
    in_specs = [off_blk, blk, row, full, full, full, full, full,
                row, row, row, row, row, row]
    operands = [e, g, t_emb, P_w, ew1, ew2, tw1, tw2, en_g, en_b, eb1,
                eb2, tb1, tb2]
    aliases = {}
    if e_new_prev is not None:
        in_specs.append(pl.BlockSpec((8, D), lambda i: (0, 0)))
        operands.append(e_new_prev)
        aliases = {14: 0}
    return pl.pallas_call(
        body,
        grid=(nb,),
        in_specs=in_specs,
        out_specs=[off_blk, blk],
        out_shape=[
            jax.ShapeDtypeStruct((E, D), jnp.float32),
            jax.ShapeDtypeStruct((Ep, D), jnp.float32),
        ],
        input_output_aliases=aliases,
    )(*operands)


def _node_update(h, uh, parts, nn_g, nn_b):
    N, D = h.shape
    nb = 10
    bl = N // nb
    np_ = len(parts)

    def body(h_ref, uh_ref, *rest):
        a_refs = rest[:np_]
        g_ref, b_ref, o_ref = rest[np_], rest[np_ + 1], rest[np_ + 2]
        x = uh_ref[...]
        for a in a_refs:
            x = x + a[...]
        o_ref[...] = h_ref[...] + jnp.maximum(
            _ln_block(x, g_ref[...], b_ref[...]), 0.0)

    blk = pl.BlockSpec((bl, D), lambda i: (i, 0))
    row = pl.BlockSpec((1, D), lambda i: (0, 0))
    return pl.pallas_call(
        body,
        grid=(nb,),
        in_specs=[blk, blk] + [blk] * np_ + [row, row],
        out_specs=blk,
        out_shape=jax.ShapeDtypeStruct((N, D), jnp.float32),
    )(h, uh, *parts, nn_g, nn_b)


def kernel(h, e, edge_index, t_emb, P_w, Q_w, R_w, en_g, en_b, ew1, eb1, ew2,
           eb2, tw1, tb1, tw2, tb2, U_w, V_w, nn_g, nn_b):
    N, D = h.shape
    E = e.shape[0]
    EB = 2560

    src = edge_index[0]
    dst = edge_index[1]

    w4 = jnp.concatenate([Q_w, R_w, V_w, U_w], axis=1)
    hq, hr, vh, uh = _node_mm(h, w4)

    E0 = (E // 2 + NW * 128 - 1) // (NW * 128) * (NW * 128)
    assert E0 % EB == 0 and (E - E0) % EB == 0
    bounds = [(0, E0), (E0, E)]

    rows_per_tile = ((N + NS - 1) // NS + 7) // 8 * 8
    zrows = jnp.zeros((rows_per_tile, D), jnp.float32)

    gs = []
    for lo, hi in bounds:
        gs.append(_make_gather_add(N, hi - lo, D, 128)(
            hq, hr, src[lo:hi], dst[lo:hi]))

    e_new = None
    gates = []
    for (lo, hi), g in zip(bounds, gs):
        e_new, gate = _edge_mlp(
            e, g, t_emb, P_w, ew1, ew2, tw1, tw2,
            en_g.reshape(1, D), en_b.reshape(1, D), eb1.reshape(1, D),
            eb2.reshape(1, D), tb1.reshape(1, D), tb2.reshape(1, D),
            lo // EB, e_new)
        gates.append(gate)

    parts = []
    for (lo, hi), gate in zip(bounds, gates):
        aggp = _make_scatter_agg(N, hi - lo, D, 64)(
            gate, vh, src[lo:hi], dst[lo:hi], zrows)
        parts.extend([aggp[0, :N], aggp[1, :N]])

    h_new = _node_update(h, uh, parts, nn_g.reshape(1, D), nn_b.reshape(1, D))
    return (h_new, e_new)


# speedup vs baseline: 5.7691x; 1.0004x over previous
import functools

import jax
import jax.numpy as jnp
from jax import lax
from jax.experimental import pallas as pl
from jax.experimental.pallas import tpu as pltpu
from jax.experimental.pallas import tpu_sc as plsc

NC = 2
NS = 16
NW = NC * NS
LANES = 16


@functools.lru_cache(maxsize=None)
def _make_gather_add(N, E, D, CH):
    per_w = E // NW
    nfull = per_w // CH
    rem = per_w - nfull * CH
    assert nfull >= 2 and nfull % 2 == 0
    mesh = plsc.VectorSubcoreMesh(core_axis_name="c", subcore_axis_name="s")

    assert nfull >= 6

    @functools.partial(
        pl.kernel,
        out_type=jax.ShapeDtypeStruct((E, D), jnp.float32),
        mesh=mesh,
        scratch_types=[
            pltpu.VMEM((per_w,), jnp.int32),
            pltpu.VMEM((per_w,), jnp.int32),
            pltpu.VMEM((CH, D), jnp.float32),
            pltpu.VMEM((CH, D), jnp.float32),
            pltpu.VMEM((CH, D), jnp.float32),
            pltpu.VMEM((CH, D), jnp.float32),
            pltpu.VMEM((CH, D), jnp.float32),
            pltpu.VMEM((CH, D), jnp.float32),
            pltpu.VMEM((max(rem, 1), D), jnp.float32),
            pltpu.VMEM((max(rem, 1), D), jnp.float32),
            pltpu.SemaphoreType.DMA,
            pltpu.SemaphoreType.DMA,
            pltpu.SemaphoreType.DMA,
            pltpu.SemaphoreType.DMA,
            pltpu.SemaphoreType.DMA,
            pltpu.SemaphoreType.DMA,
        ],
    )
    def k(hq, hr, src, dst, g, sall, dall, q0, r0, q1, r1, q2, r2,
          qv2, rv2, sg0, sg1, sg2, ss0, ss1, ss2):
        c = lax.axis_index("c")
        s = lax.axis_index("s")
        base = (c * NS + s) * per_w
        qs = (q0, q1, q2)
        rs = (r0, r1, r2)
        sgs = (sg0, sg1, sg2)
        sss = (ss0, ss1, ss2)

        pltpu.sync_copy(src.at[pl.ds(base, per_w)], sall)
        pltpu.sync_copy(dst.at[pl.ds(base, per_w)], dall)

        def fire(cj, b):
            isl = sall.at[pl.ds(cj * CH, CH)]
            idl = dall.at[pl.ds(cj * CH, CH)]
            pltpu.async_copy(hq.at[isl], qs[b], sgs[b])
            pltpu.async_copy(hr.at[idl], rs[b], sgs[b])

        def wait_gathers(b):
            pltpu.make_async_copy(hq.at[sall.at[pl.ds(0, CH)]], qs[b],
                                  sgs[b]).wait()
            pltpu.make_async_copy(hr.at[dall.at[pl.ds(0, CH)]], rs[b],
                                  sgs[b]).wait()

        def wait_store(b):
            pltpu.make_async_copy(qs[b], g.at[pl.ds(base, CH)], sss[b]).wait()

        def add_rows(qr, rr, n):
            def row(i, _):
                for gi in range(D // LANES):
                    sl = pl.ds(gi * LANES, LANES)
                    qr[i, sl] = qr[i, sl] + rr[i, sl]
                return 0

            lax.fori_loop(0, n, row, 0)

        def store(cj, b):
            pltpu.async_copy(qs[b], g.at[pl.ds(base + cj * CH, CH)], sss[b])

        def process(cj, b):
            wait_gathers(b)
            add_rows(qs[b], rs[b], CH)
            store(cj, b)

        fire(0, 0)
        fire(1, 1)
        fire(2, 2)
        process(0, 0)

        m = (nfull - 3) // 3

        def body(t, _):
            c0 = 3 * t + 1
            for kk in range(3):
                bcur = (1 + kk) % 3
                bfire = kk % 3
                wait_store(bfire)
                fire(c0 + kk + 2, bfire)
                process(c0 + kk, bcur)
            return 0

        lax.fori_loop(0, m, body, 0)
        for cj in range(3 * m + 1, nfull):
            if cj <= nfull - 3:
                wait_store((cj + 2) % 3)
                fire(cj + 2, (cj + 2) % 3)
            process(cj, cj % 3)
        for cj in range(nfull - 3, nfull):
            wait_store(cj % 3)

        if rem:
            off = nfull * CH
            isl = sall.at[pl.ds(off, rem)]
            idl = dall.at[pl.ds(off, rem)]
            cq = pltpu.async_copy(hq.at[isl], qv2, sg0)
            cr = pltpu.async_copy(hr.at[idl], rv2, sg0)
            cq.wait()
            cr.wait()
            add_rows(qv2, rv2, rem)
            pltpu.sync_copy(qv2, g.at[pl.ds(base + off, rem)])

    return k


@functools.lru_cache(maxsize=None)
def _make_scatter_agg(N, E, D, CH):
    per_w = E // NW
    nfull = per_w // CH
    rem = per_w - nfull * CH
    rows_per_tile = ((N + NS - 1) // NS + 7) // 8 * 8
    npad = rows_per_tile * NS
    mesh = plsc.VectorSubcoreMesh(core_axis_name="c", subcore_axis_name="s")

    assert nfull >= 2 and nfull % 2 == 0

    @functools.partial(
        pl.kernel,
        out_type=jax.ShapeDtypeStruct((NC, npad, D), jnp.float32),
        mesh=mesh,
        scratch_types=[
            pltpu.VMEM((per_w,), jnp.int32),
            pltpu.VMEM((CH,), jnp.int32),
            pltpu.VMEM((CH,), jnp.int32),
            pltpu.VMEM((CH, D), jnp.float32),
            pltpu.VMEM((CH, D), jnp.float32),
            pltpu.VMEM((CH, D), jnp.float32),
            pltpu.VMEM((CH, D), jnp.float32),
            pltpu.VMEM((max(rem, 1),), jnp.int32),
            pltpu.VMEM((max(rem, 1), D), jnp.float32),
            pltpu.VMEM((max(rem, 1), D), jnp.float32),
            pltpu.VMEM_SHARED((npad, D), jnp.float32),
            pltpu.SemaphoreType.DMA,
            pltpu.SemaphoreType.DMA,
            pltpu.SemaphoreType.DMA,
            pltpu.SemaphoreType.DMA,
        ],
    )
    def k(gate, vh, src, dst, zrows, out, dall, s0, s1, g0, v0, g1, v1,
          isv2, gv2, vv2, agg, sg0, sg1, ss0, ss1):
        c = lax.axis_index("c")
        s = lax.axis_index("s")
        base = (c * NS + s) * per_w
        svs = (s0, s1)
        gs = (g0, g1)
        vs = (v0, v1)
        sgs = (sg0, sg1)
        sss = (ss0, ss1)

        pltpu.sync_copy(zrows, agg.at[pl.ds(s * rows_per_tile, rows_per_tile)])
        pltpu.sync_copy(dst.at[pl.ds(base, per_w)], dall)
        plsc.subcore_barrier()

        def fire(cj, b):
            off = base + cj * CH
            idl = dall.at[pl.ds(cj * CH, CH)]
            pltpu.async_copy(src.at[pl.ds(off, CH)], svs[b], sgs[b])
            pltpu.async_copy(gate.at[pl.ds(off, CH)], gs[b], sgs[b])
            pltpu.async_copy(vh.at[idl], vs[b], sgs[b])

        def wait_fire(b):
            pltpu.make_async_copy(src.at[pl.ds(base, CH)], svs[b],
                                  sgs[b]).wait()
            pltpu.make_async_copy(gate.at[pl.ds(base, CH)], gs[b],
                                  sgs[b]).wait()
            pltpu.make_async_copy(vh.at[dall.at[pl.ds(0, CH)]], vs[b],
                                  sgs[b]).wait()

        def mul_rows(gr, vr, n):
            def row(i, _):
                for gi in range(D // LANES):
                    sl = pl.ds(gi * LANES, LANES)
                    gr[i, sl] = gr[i, sl] * vr[i, sl]
                return 0

            lax.fori_loop(0, n, row, 0)

        def scatter(b):
            pltpu.async_copy(gs[b], agg.at[svs[b]], sss[b], add=True)

        def wait_scatter(b):
            pltpu.make_async_copy(gs[b], agg.at[svs[b]], sss[b]).wait()

        fire(0, 0)
        fire(1, 1)
        wait_fire(0)
        mul_rows(g0, v0, CH)
        scatter(0)

        def body(i, _):
            j = 2 * i + 1
            wait_scatter(0)
            fire(j + 1, 0)
            wait_fire(1)
            mul_rows(g1, v1, CH)
            scatter(1)

            wait_scatter(1)
            fire(j + 2, 1)
            wait_fire(0)
            mul_rows(g0, v0, CH)
            scatter(0)
            return 0

        lax.fori_loop(0, (nfull - 2) // 2, body, 0)
        wait_scatter(0)
        wait_fire(1)
        mul_rows(g1, v1, CH)
        scatter(1)
        wait_scatter(1)

        if rem:
            off = base + nfull * CH
            ci = pltpu.async_copy(src.at[pl.ds(off, rem)], isv2, sg0)
            cg = pltpu.async_copy(gate.at[pl.ds(off, rem)], gv2, sg0)
            cv = pltpu.async_copy(vh.at[dall.at[pl.ds(nfull * CH, rem)]],
                                  vv2, sg0)
            ci.wait()
            cg.wait()
            cv.wait()
            mul_rows(gv2, vv2, rem)
            pltpu.sync_copy(gv2, agg.at[isv2], add=True)

        plsc.subcore_barrier()
        pltpu.sync_copy(
            agg.at[pl.ds(s * rows_per_tile, rows_per_tile)],
            out.at[c, pl.ds(s * rows_per_tile, rows_per_tile)],
        )

    return k


def _node_mm(h, w4):
    N, D = h.shape
    D4 = w4.shape[1]
    nb = 10
    bl = N // nb

    def body(h_ref, w_ref, hq_ref, hr_ref, vh_ref, uh_ref):
        z = jnp.dot(h_ref[...], w_ref[...], preferred_element_type=jnp.float32)
        hq_ref[...] = z[:, 0:D]
        hr_ref[...] = z[:, D:2 * D]
        vh_ref[...] = z[:, 2 * D:3 * D]
        uh_ref[...] = z[:, 3 * D:4 * D]

    blk = pl.BlockSpec((bl, D), lambda i: (i, 0))
    sds = jax.ShapeDtypeStruct((N, D), jnp.float32)
    return pl.pallas_call(
        body,
        grid=(nb,),
        in_specs=[
            pl.BlockSpec((bl, D), lambda i: (i, 0)),
            pl.BlockSpec((D, D4), lambda i: (0, 0)),
        ],
        out_specs=[blk, blk, blk, blk],
        out_shape=[sds, sds, sds, sds],
    )(h, w4)


def _ln_block(x, g, b, eps=1e-5):
    m = jnp.mean(x, axis=-1, keepdims=True)
    cx = x - m
    v = jnp.mean(cx * cx, axis=-1, keepdims=True)
    return cx / jnp.sqrt(v + eps) * g + b


def _edge_mlp(e, g, t_emb, P_w, ew1, ew2, tw1, tw2, en_g, en_b, eb1, eb2,
              tb1, tb2, blk_off, e_new_prev):
    E, D = e.shape
    Ep = g.shape[0]
    eb_blk = 2560
    nb = Ep // eb_blk

    def body(e_ref, g_ref, t_ref, pw, w1, w2, tw1r, tw2r, eng, enb,
             b1, b2, tb1r, tb2r, *rest):
        enew_ref, gate_ref = rest[-2], rest[-1]
        eb = e_ref[...]
        e_hat = jnp.dot(eb, pw[...], preferred_element_type=jnp.float32) \
            + g_ref[...]
        xn = _ln_block(e_hat, eng[...], enb[...])
        h1 = jnp.maximum(
            jnp.dot(xn, w1[...], preferred_element_type=jnp.float32) + b1[...],
            0.0)
        mlp_e = jnp.dot(h1, w2[...], preferred_element_type=jnp.float32) \
            + b2[...]
        t1 = jnp.maximum(
            jnp.dot(t_ref[...], tw1r[...], preferred_element_type=jnp.float32)
            + tb1r[...], 0.0)
        mlp_t = jnp.dot(t1, tw2r[...], preferred_element_type=jnp.float32) \
            + tb2r[...]
        enew_ref[...] = eb + mlp_e + mlp_t
        gate_ref[...] = jax.nn.sigmoid(e_hat)

    full = pl.BlockSpec((D, D), lambda i: (0, 0))
    row = pl.BlockSpec((1, D), lambda i: (0, 0))
    blk = pl.BlockSpec((eb_blk, D), lambda i: (i, 0))
    off_blk = pl.BlockSpec((eb_blk, D), lambda i: (i + blk_off, 0))
    in_specs = [off_blk, blk, row, full, full, full, full, full,
                row, row, row, row, row, row]
    operands = [e, g, t_emb, P_w, ew1, ew2, tw1, tw2, en_g, en_b, eb1,
                eb2, tb1, tb2]
    aliases = {}
    if e_new_prev is not None:
        in_specs.append(pl.BlockSpec((8, D), lambda i: (0, 0)))
        operands.append(e_new_prev)
        aliases = {14: 0}
    return pl.pallas_call(
        body,
        grid=(nb,),
        in_specs=in_specs,
        out_specs=[off_blk, blk],
        out_shape=[
            jax.ShapeDtypeStruct((E, D), jnp.float32),
            jax.ShapeDtypeStruct((Ep, D), jnp.float32),
        ],
        input_output_aliases=aliases,
    )(*operands)


def _node_update(h, uh, parts, nn_g, nn_b):
    N, D = h.shape
    nb = 10
    bl = N // nb
    np_ = len(parts)

    def body(h_ref, uh_ref, *rest):
        a_refs = rest[:np_]
        g_ref, b_ref, o_ref = rest[np_], rest[np_ + 1], rest[np_ + 2]
        x = uh_ref[...]
        for a in a_refs:
            x = x + a[...]
        o_ref[...] = h_ref[...] + jnp.maximum(
            _ln_block(x, g_ref[...], b_ref[...]), 0.0)

    blk = pl.BlockSpec((bl, D), lambda i: (i, 0))
    row = pl.BlockSpec((1, D), lambda i: (0, 0))
    return pl.pallas_call(
        body,
        grid=(nb,),
        in_specs=[blk, blk] + [blk] * np_ + [row, row],
        out_specs=blk,
        out_shape=jax.ShapeDtypeStruct((N, D), jnp.float32),
    )(h, uh, *parts, nn_g, nn_b)


def kernel(h, e, edge_index, t_emb, P_w, Q_w, R_w, en_g, en_b, ew1, eb1, ew2,
           eb2, tw1, tb1, tw2, tb2, U_w, V_w, nn_g, nn_b):
    N, D = h.shape
    E = e.shape[0]
    EB = 2560

    src = edge_index[0]
    dst = edge_index[1]

    w4 = jnp.concatenate([Q_w, R_w, V_w, U_w], axis=1)
    hq, hr, vh, uh = _node_mm(h, w4)

    E0 = (E // 2 + NW * 128 - 1) // (NW * 128) * (NW * 128)
    assert E0 % EB == 0 and (E - E0) % EB == 0
    bounds = [(0, E0), (E0, E)]

    rows_per_tile = ((N + NS - 1) // NS + 7) // 8 * 8
    zrows = jnp.zeros((rows_per_tile, D), jnp.float32)

    gs = []
    for lo, hi in bounds:
        gs.append(_make_gather_add(N, hi - lo, D, 128)(
            hq, hr, src[lo:hi], dst[lo:hi]))

    e_new = None
    gates = []
    for (lo, hi), g in zip(bounds, gs):
        e_new, gate = _edge_mlp(
            e, g, t_emb, P_w, ew1, ew2, tw1, tw2,
            en_g.reshape(1, D), en_b.reshape(1, D), eb1.reshape(1, D),
            eb2.reshape(1, D), tb1.reshape(1, D), tb2.reshape(1, D),
            lo // EB, e_new)
        gates.append(gate)

    parts = []
    for (lo, hi), gate in zip(bounds, gates):
        aggp = _make_scatter_agg(N, hi - lo, D, 64)(
            gate, vh, src[lo:hi], dst[lo:hi], zrows)
        parts.extend([aggp[0, :N], aggp[1, :N]])

    h_new = _node_update(h, uh, parts, nn_g.reshape(1, D), nn_b.reshape(1, D))
    return (h_new, e_new)


# speedup vs baseline: 5.8789x; 1.0190x over previous
import functools

import jax
import jax.numpy as jnp
from jax import lax
from jax.experimental import pallas as pl
from jax.experimental.pallas import tpu as pltpu
from jax.experimental.pallas import tpu_sc as plsc

NC = 2
NS = 16
NW = NC * NS
LANES = 16


@functools.lru_cache(maxsize=None)
def _make_gather_add(N, E, D, CH):
    per_w = E // NW
    nfull = per_w // CH
    rem = per_w - nfull * CH
    assert nfull >= 2 and nfull % 2 == 0
    mesh = plsc.VectorSubcoreMesh(core_axis_name="c", subcore_axis_name="s")

    assert nfull >= 6

    @functools.partial(
        pl.kernel,
        out_type=jax.ShapeDtypeStruct((E, D), jnp.float32),
        mesh=mesh,
        scratch_types=[
            pltpu.VMEM((per_w,), jnp.int32),
            pltpu.VMEM((per_w,), jnp.int32),
            pltpu.VMEM((CH, D), jnp.float32),
            pltpu.VMEM((CH, D), jnp.float32),
            pltpu.VMEM((CH, D), jnp.float32),
            pltpu.VMEM((CH, D), jnp.float32),
            pltpu.VMEM((CH, D), jnp.float32),
            pltpu.VMEM((CH, D), jnp.float32),
            pltpu.VMEM((max(rem, 1), D), jnp.float32),
            pltpu.VMEM((max(rem, 1), D), jnp.float32),
            pltpu.SemaphoreType.DMA,
            pltpu.SemaphoreType.DMA,
            pltpu.SemaphoreType.DMA,
            pltpu.SemaphoreType.DMA,
            pltpu.SemaphoreType.DMA,
            pltpu.SemaphoreType.DMA,
        ],
    )
    def k(hq, hr, src, dst, g, sall, dall, q0, r0, q1, r1, q2, r2,
          qv2, rv2, sg0, sg1, sg2, ss0, ss1, ss2):
        c = lax.axis_index("c")
        s = lax.axis_index("s")
        base = (c * NS + s) * per_w
        qs = (q0, q1, q2)
        rs = (r0, r1, r2)
        sgs = (sg0, sg1, sg2)
        sss = (ss0, ss1, ss2)

        pltpu.sync_copy(src.at[pl.ds(base, per_w)], sall)
        pltpu.sync_copy(dst.at[pl.ds(base, per_w)], dall)

        def fire(cj, b):
            isl = sall.at[pl.ds(cj * CH, CH)]
            idl = dall.at[pl.ds(cj * CH, CH)]
            pltpu.async_copy(hq.at[isl], qs[b], sgs[b])
            pltpu.async_copy(hr.at[idl], rs[b], sgs[b])

        def wait_gathers(b):
            pltpu.make_async_copy(hq.at[sall.at[pl.ds(0, CH)]], qs[b],
                                  sgs[b]).wait()
            pltpu.make_async_copy(hr.at[dall.at[pl.ds(0, CH)]], rs[b],
                                  sgs[b]).wait()

        def wait_store(b):
            pltpu.make_async_copy(qs[b], g.at[pl.ds(base, CH)], sss[b]).wait()

        def add_rows(qr, rr, n):
            def row(i, _):
                for gi in range(D // LANES):
                    sl = pl.ds(gi * LANES, LANES)
                    qr[i, sl] = qr[i, sl] + rr[i, sl]
                return 0

            lax.fori_loop(0, n, row, 0)

        def store(cj, b):
            pltpu.async_copy(qs[b], g.at[pl.ds(base + cj * CH, CH)], sss[b])

        def process(cj, b):
            wait_gathers(b)
            add_rows(qs[b], rs[b], CH)
            store(cj, b)

        fire(0, 0)
        fire(1, 1)
        fire(2, 2)
        process(0, 0)

        m = (nfull - 3) // 3

        def body(t, _):
            c0 = 3 * t + 1
            for kk in range(3):
                bcur = (1 + kk) % 3
                bfire = kk % 3
                wait_store(bfire)
                fire(c0 + kk + 2, bfire)
                process(c0 + kk, bcur)
            return 0

        lax.fori_loop(0, m, body, 0)
        for cj in range(3 * m + 1, nfull):
            if cj <= nfull - 3:
                wait_store((cj + 2) % 3)
                fire(cj + 2, (cj + 2) % 3)
            process(cj, cj % 3)
        for cj in range(nfull - 3, nfull):
            wait_store(cj % 3)

        if rem:
            off = nfull * CH
            isl = sall.at[pl.ds(off, rem)]
            idl = dall.at[pl.ds(off, rem)]
            cq = pltpu.async_copy(hq.at[isl], qv2, sg0)
            cr = pltpu.async_copy(hr.at[idl], rv2, sg0)
            cq.wait()
            cr.wait()
            add_rows(qv2, rv2, rem)
            pltpu.sync_copy(qv2, g.at[pl.ds(base + off, rem)])

    return k


@functools.lru_cache(maxsize=None)
def _make_scatter_agg(N, E, D, CH):
    per_w = E // NW
    nfull = per_w // CH
    rem = per_w - nfull * CH
    rows_per_tile = ((N + NS - 1) // NS + 7) // 8 * 8
    npad = rows_per_tile * NS
    mesh = plsc.VectorSubcoreMesh(core_axis_name="c", subcore_axis_name="s")

    assert nfull >= 2 and nfull % 2 == 0

    @functools.partial(
        pl.kernel,
        out_type=jax.ShapeDtypeStruct((NC, npad, D), jnp.float32),
        mesh=mesh,
        scratch_types=[
            pltpu.VMEM((per_w,), jnp.int32),
            pltpu.VMEM((CH,), jnp.int32),
            pltpu.VMEM((CH,), jnp.int32),
            pltpu.VMEM((CH, D), jnp.float32),
            pltpu.VMEM((CH, D), jnp.float32),
            pltpu.VMEM((CH, D), jnp.float32),
            pltpu.VMEM((CH, D), jnp.float32),
            pltpu.VMEM((max(rem, 1),), jnp.int32),
            pltpu.VMEM((max(rem, 1), D), jnp.float32),
            pltpu.VMEM((max(rem, 1), D), jnp.float32),
            pltpu.VMEM_SHARED((npad, D), jnp.float32),
            pltpu.SemaphoreType.DMA,
            pltpu.SemaphoreType.DMA,
            pltpu.SemaphoreType.DMA,
            pltpu.SemaphoreType.DMA,
        ],
    )
    def k(gate, vh, src, dst, zrows, out, dall, s0, s1, g0, v0, g1, v1,
          isv2, gv2, vv2, agg, sg0, sg1, ss0, ss1):
        c = lax.axis_index("c")
        s = lax.axis_index("s")
        base = (c * NS + s) * per_w
        svs = (s0, s1)
        gs = (g0, g1)
        vs = (v0, v1)
        sgs = (sg0, sg1)
        sss = (ss0, ss1)

        pltpu.sync_copy(zrows, agg.at[pl.ds(s * rows_per_tile, rows_per_tile)])
        pltpu.sync_copy(dst.at[pl.ds(base, per_w)], dall)
        plsc.subcore_barrier()

        def fire(cj, b):
            off = base + cj * CH
            idl = dall.at[pl.ds(cj * CH, CH)]
            pltpu.async_copy(src.at[pl.ds(off, CH)], svs[b], sgs[b])
            pltpu.async_copy(gate.at[pl.ds(off, CH)], gs[b], sgs[b])
            pltpu.async_copy(vh.at[idl], vs[b], sgs[b])

        def wait_fire(b):
            pltpu.make_async_copy(src.at[pl.ds(base, CH)], svs[b],
                                  sgs[b]).wait()
            pltpu.make_async_copy(gate.at[pl.ds(base, CH)], gs[b],
                                  sgs[b]).wait()
            pltpu.make_async_copy(vh.at[dall.at[pl.ds(0, CH)]], vs[b],
                                  sgs[b]).wait()

        def mul_rows(gr, vr, n):
            def row(i, _):
                for gi in range(D // LANES):
                    sl = pl.ds(gi * LANES, LANES)
                    gr[i, sl] = gr[i, sl] * vr[i, sl]
                return 0

            lax.fori_loop(0, n, row, 0)

        def scatter(b):
            pltpu.async_copy(gs[b], agg.at[svs[b]], sss[b], add=True)

        def wait_scatter(b):
            pltpu.make_async_copy(gs[b], agg.at[svs[b]], sss[b]).wait()

        fire(0, 0)
        fire(1, 1)
        wait_fire(0)
        mul_rows(g0, v0, CH)
        scatter(0)

        def body(i, _):
            j = 2 * i + 1
            wait_scatter(0)
            fire(j + 1, 0)
            wait_fire(1)
            mul_rows(g1, v1, CH)
            scatter(1)

            wait_scatter(1)
            fire(j + 2, 1)
            wait_fire(0)
            mul_rows(g0, v0, CH)
            scatter(0)
            return 0

        lax.fori_loop(0, (nfull - 2) // 2, body, 0)
        wait_scatter(0)
        wait_fire(1)
        mul_rows(g1, v1, CH)
        scatter(1)
        wait_scatter(1)

        if rem:
            off = base + nfull * CH
            ci = pltpu.async_copy(src.at[pl.ds(off, rem)], isv2, sg0)
            cg = pltpu.async_copy(gate.at[pl.ds(off, rem)], gv2, sg0)
            cv = pltpu.async_copy(vh.at[dall.at[pl.ds(nfull * CH, rem)]],
                                  vv2, sg0)
            ci.wait()
            cg.wait()
            cv.wait()
            mul_rows(gv2, vv2, rem)
            pltpu.sync_copy(gv2, agg.at[isv2], add=True)

        plsc.subcore_barrier()
        pltpu.sync_copy(
            agg.at[pl.ds(s * rows_per_tile, rows_per_tile)],
            out.at[c, pl.ds(s * rows_per_tile, rows_per_tile)],
        )

    return k


def _node_mm(h, w4):
    N, D = h.shape
    D4 = w4.shape[1]
    nb = 10
    bl = N // nb

    def body(h_ref, w_ref, hq_ref, hr_ref, vh_ref, uh_ref):
        z = jnp.dot(h_ref[...], w_ref[...], preferred_element_type=jnp.float32)
        hq_ref[...] = z[:, 0:D]
        hr_ref[...] = z[:, D:2 * D]
        vh_ref[...] = z[:, 2 * D:3 * D]
        uh_ref[...] = z[:, 3 * D:4 * D]

    blk = pl.BlockSpec((bl, D), lambda i: (i, 0))
    sds = jax.ShapeDtypeStruct((N, D), jnp.float32)
    return pl.pallas_call(
        body,
        grid=(nb,),
        in_specs=[
            pl.BlockSpec((bl, D), lambda i: (i, 0)),
            pl.BlockSpec((D, D4), lambda i: (0, 0)),
        ],
        out_specs=[blk, blk, blk, blk],
        out_shape=[sds, sds, sds, sds],
    )(h, w4)


def _ln_block(x, g, b, eps=1e-5):
    m = jnp.mean(x, axis=-1, keepdims=True)
    cx = x - m
    v = jnp.mean(cx * cx, axis=-1, keepdims=True)
    return cx / jnp.sqrt(v + eps) * g + b


def _edge_mlp(e, g, t_emb, P_w, ew1, ew2, tw1, tw2, en_g, en_b, eb1, eb2,
              tb1, tb2, blk_off, e_new_prev):
    E, D = e.shape
    Ep = g.shape[0]
    eb_blk = 2560
    nb = Ep // eb_blk

    def body(e_ref, g_ref, t_ref, pw, w1, w2, tw1r, tw2r, eng, enb,
             b1, b2, tb1r, tb2r, *rest):
        enew_ref, gate_ref = rest[-2], rest[-1]
        eb = e_ref[...]
        e_hat = jnp.dot(eb, pw[...], preferred_element_type=jnp.float32) \
            + g_ref[...]
        xn = _ln_block(e_hat, eng[...], enb[...])
        h1 = jnp.maximum(
            jnp.dot(xn, w1[...], preferred_element_type=jnp.float32) + b1[...],
            0.0)
        mlp_e = jnp.dot(h1, w2[...], preferred_element_type=jnp.float32) \
            + b2[...]
        t1 = jnp.maximum(
            jnp.dot(t_ref[...], tw1r[...], preferred_element_type=jnp.float32)
            + tb1r[...], 0.0)
        mlp_t = jnp.dot(t1, tw2r[...], preferred_element_type=jnp.float32) \
            + tb2r[...]
        enew_ref[...] = eb + mlp_e + mlp_t
        gate_ref[...] = jax.nn.sigmoid(e_hat)

    full = pl.BlockSpec((D, D), lambda i: (0, 0))
    row = pl.BlockSpec((1, D), lambda i: (0, 0))
    blk = pl.BlockSpec((eb_blk, D), lambda i: (i, 0))
    off_blk = pl.BlockSpec((eb_blk, D), lambda i: (i + blk_off, 0))
    in_specs = [off_blk, blk, row, full, full, full, full, full,
                row, row, row, row, row, row]
    operands = [e, g, t_emb, P_w, ew1, ew2, tw1, tw2, en_g, en_b, eb1,
                eb2, tb1, tb2]
    aliases = {}
    if e_new_prev is not None:
        in_specs.append(pl.BlockSpec((8, D), lambda i: (0, 0)))
        operands.append(e_new_prev)
        aliases = {14: 0}
    return pl.pallas_call(
        body,
        grid=(nb,),
        in_specs=in_specs,
        out_specs=[off_blk, blk],
        out_shape=[
            jax.ShapeDtypeStruct((E, D), jnp.float32),
            jax.ShapeDtypeStruct((Ep, D), jnp.float32),
        ],
        input_output_aliases=aliases,
    )(*operands)


def _node_update(h, uh, parts, nn_g, nn_b):
    N, D = h.shape
    nb = 10
    bl = N // nb
    np_ = len(parts)

    def body(h_ref, uh_ref, *rest):
        a_refs = rest[:np_]
        g_ref, b_ref, o_ref = rest[np_], rest[np_ + 1], rest[np_ + 2]
        x = uh_ref[...]
        for a in a_refs:
            x = x + a[0] + a[1]
        o_ref[...] = h_ref[...] + jnp.maximum(
            _ln_block(x, g_ref[...], b_ref[...]), 0.0)

    blk = pl.BlockSpec((bl, D), lambda i: (i, 0))
    pblk = pl.BlockSpec((NC, bl, D), lambda i: (0, i, 0))
    row = pl.BlockSpec((1, D), lambda i: (0, 0))
    return pl.pallas_call(
        body,
        grid=(nb,),
        in_specs=[blk, blk] + [pblk] * np_ + [row, row],
        out_specs=blk,
        out_shape=jax.ShapeDtypeStruct((N, D), jnp.float32),
    )(h, uh, *parts, nn_g, nn_b)


def kernel(h, e, edge_index, t_emb, P_w, Q_w, R_w, en_g, en_b, ew1, eb1, ew2,
           eb2, tw1, tb1, tw2, tb2, U_w, V_w, nn_g, nn_b):
    N, D = h.shape
    E = e.shape[0]
    EB = 2560

    src = edge_index[0]
    dst = edge_index[1]

    w4 = jnp.concatenate([Q_w, R_w, V_w, U_w], axis=1)
    hq, hr, vh, uh = _node_mm(h, w4)

    E0 = (E // 2 + NW * 128 - 1) // (NW * 128) * (NW * 128)
    assert E0 % EB == 0 and (E - E0) % EB == 0
    bounds = [(0, E0), (E0, E)]

    rows_per_tile = ((N + NS - 1) // NS + 7) // 8 * 8
    zrows = jnp.zeros((rows_per_tile, D), jnp.float32)

    gs = []
    for lo, hi in bounds:
        gs.append(_make_gather_add(N, hi - lo, D, 128)(
            hq, hr, src[lo:hi], dst[lo:hi]))

    e_new = None
    gates = []
    for (lo, hi), g in zip(bounds, gs):
        e_new, gate = _edge_mlp(
            e, g, t_emb, P_w, ew1, ew2, tw1, tw2,
            en_g.reshape(1, D), en_b.reshape(1, D), eb1.reshape(1, D),
            eb2.reshape(1, D), tb1.reshape(1, D), tb2.reshape(1, D),
            lo // EB, e_new)
        gates.append(gate)

    parts = []
    for (lo, hi), gate in zip(bounds, gates):
        aggp = _make_scatter_agg(N, hi - lo, D, 64)(
            gate, vh, src[lo:hi], dst[lo:hi], zrows)
        parts.append(aggp)

    h_new = _node_update(h, uh, parts, nn_g.reshape(1, D), nn_b.reshape(1, D))
    return (h_new, e_new)
